# baseline (speedup 1.0000x reference)
import functools

import jax
import jax.numpy as jnp
from jax import lax
from jax.experimental import pallas as pl
from jax.experimental.pallas import tpu as pltpu
from jax.experimental.pallas import tpu_sc as plsc

f32 = jnp.float32

D = 128
NC, NS, L = 2, 16, 16
NW = NC * NS
CHUNK = 128
NPAD = 10240
SLAB = NPAD // NS
N = 10000
TRASH = N


def _sc_body(with_cnt, nch, *refs):
  if with_cnt:
    (h, srci, dsti, zrows, orows,
     out, cntout,
     vsrc8, vdst8, rows0, acc, g0, g1) = refs
  else:
    (h, srci, dsti, zrows,
     out,
     vsrc8, vdst8, rows0, acc, g0, g1) = refs
    cntout = orows = None
  c = lax.axis_index("c")
  s = lax.axis_index("s")
  t = c * NS + s
  n = nch

  pltpu.sync_copy(zrows, rows0)

  def zbody(j, carry):
    r = s * SLAB + j * CHUNK
    pltpu.sync_copy(rows0, acc.at[pl.ds(r, CHUNK)])
    return carry

  lax.fori_loop(0, SLAB // CHUNK, zbody, 0)
  plsc.subcore_barrier()

  def pbody(i, carry):
    r0 = t * nch + i * 8
    pltpu.sync_copy(srci.at[pl.ds(r0, 8)], vsrc8)
    pltpu.sync_copy(dsti.at[pl.ds(r0, 8)], vdst8)
    for k in range(8):
      pltpu.async_copy(h.at[vsrc8.at[k]], rows0, g1).wait()
      pltpu.sync_copy(rows0, acc.at[vdst8.at[k]], add=True)
    return carry

  lax.fori_loop(0, n // 8, pbody, 0)
  plsc.subcore_barrier()

  def obody(dst_hbm, j, carry):
    r = s * SLAB + j * CHUNK
    pltpu.sync_copy(acc.at[pl.ds(r, CHUNK)], rows0)
    pltpu.sync_copy(rows0, dst_hbm.at[c, pl.ds(r, CHUNK)])
    return carry

  lax.fori_loop(0, SLAB // CHUNK, functools.partial(obody, out), 0)

  if with_cnt:
    plsc.subcore_barrier()
    pltpu.sync_copy(zrows, rows0)
    lax.fori_loop(0, SLAB // CHUNK, zbody, 0)
    pltpu.sync_copy(orows, rows0)
    plsc.subcore_barrier()

    def cbody(i, carry):
      r0 = t * nch + i * 8
      pltpu.sync_copy(dsti.at[pl.ds(r0, 8)], vdst8)
      for k in range(8):
        pltpu.sync_copy(rows0, acc.at[vdst8.at[k]], add=True)
      return carry

    lax.fori_loop(0, n // 8, cbody, 0)
    plsc.subcore_barrier()
    lax.fori_loop(0, SLAB // CHUNK, functools.partial(obody, cntout), 0)


@functools.lru_cache(maxsize=None)
def _build_agg(epad, with_cnt):
  nch = epad // NW // CHUNK
  out_type = [jax.ShapeDtypeStruct((NC, NPAD, D), f32)]
  scratch = [
      pltpu.VMEM((8, CHUNK), jnp.int32),
      pltpu.VMEM((8, CHUNK), jnp.int32),
      pltpu.VMEM((CHUNK, D), f32),
  ]
  if with_cnt:
    out_type.append(jax.ShapeDtypeStruct((NC, NPAD, D), f32))
  scratch.append(pltpu.VMEM_SHARED((NPAD, D), f32))
  scratch += [pltpu.SemaphoreType.DMA for _ in range(2)]
  mesh = plsc.VectorSubcoreMesh(core_axis_name="c", subcore_axis_name="s")
  return pl.kernel(
      functools.partial(_sc_body, with_cnt, nch),
      out_type=out_type,
      mesh=mesh,
      scratch_types=scratch,
  )


def _tc_body(relu, p_ref, cnt_ref, h_ref, w_ref, b_ref, o_ref):
  s = p_ref[0] + p_ref[1] + h_ref[...]
  mean = s / cnt_ref[...]
  y = lax.dot_general(mean, w_ref[...], (((1,), (1,)), ((), ())),
                      preferred_element_type=f32) + b_ref[...]
  if relu:
    y = jnp.maximum(y, 0.0)
  o_ref[...] = y


BR = 2000


@functools.lru_cache(maxsize=None)
def _build_layer(relu):
  return pl.pallas_call(
      functools.partial(_tc_body, relu),
      grid=(N // BR,),
      in_specs=[
          pl.BlockSpec((NC, BR, D), lambda g: (0, g, 0)),
          pl.BlockSpec((BR, 1), lambda g: (g, 0)),
          pl.BlockSpec((BR, D), lambda g: (g, 0)),
          pl.BlockSpec((D, D), lambda g: (0, 0)),
          pl.BlockSpec((1, D), lambda g: (0, 0)),
      ],
      out_specs=pl.BlockSpec((BR, D), lambda g: (g, 0)),
      out_shape=jax.ShapeDtypeStruct((N, D), f32),
  )


def kernel(x, edge_index, W1, b1, W2, b2):
  e = edge_index.shape[1]
  step = NW * CHUNK * 8
  epad = -(-e // step) * step
  src = jnp.pad(edge_index[0].astype(jnp.int32), (0, epad - e))
  dst = jnp.pad(edge_index[1].astype(jnp.int32), (0, epad - e))
  dst = jnp.where(src == dst, TRASH, dst)
  src2 = src.reshape(-1, CHUNK)
  dst2 = dst.reshape(-1, CHUNK)
  zrows = jnp.zeros((CHUNK, D), f32)
  orows = jnp.ones((CHUNK, D), f32)

  agg1, cntg = _build_agg(epad, True)(x, src2, dst2, zrows, orows)
  cnt = (cntg[0, :N, 0] + cntg[1, :N, 0] + 1.0).reshape(N, 1)
  h1 = _build_layer(True)(agg1, cnt, x, W1, b1.reshape(1, D))
  (agg2,) = _build_agg(epad, False)(h1, src2, dst2, zrows)
  return _build_layer(False)(agg2, cnt, h1, W2, b2.reshape(1, D))


# speedup vs baseline: 1.1787x; 1.1787x over previous
import functools

import jax
import jax.numpy as jnp
from jax import lax
from jax.experimental import pallas as pl
from jax.experimental.pallas import tpu as pltpu
from jax.experimental.pallas import tpu_sc as plsc

f32 = jnp.float32

D = 128
NC, NS, L = 2, 16, 16
NW = NC * NS
CHUNK = 128
NPAD = 10240
SLAB = NPAD // NS
N = 10000
TRASH = N


def _sc_body(with_cnt, n_chunks, ept, *refs):
  if with_cnt:
    (h, srci, dsti, zrows, orows,
     out, cntout,
     vsrc, vdst2, vrows, vone, acc, sem) = refs
  else:
    (h, srci, dsti, zrows,
     out,
     vsrc, vdst2, vrows, acc, sem) = refs
    vone = cntout = orows = None
  c = lax.axis_index("c")
  s = lax.axis_index("s")
  t = c * NS + s

  pltpu.sync_copy(zrows, vrows)

  def zbody(j, carry):
    r = s * SLAB + j * CHUNK
    pltpu.sync_copy(vrows, acc.at[pl.ds(r, CHUNK)])
    return carry

  lax.fori_loop(0, SLAB // CHUNK, zbody, 0)
  if with_cnt:
    pltpu.sync_copy(orows, vone)
  plsc.subcore_barrier()

  def ebody(i, carry):
    base = t * ept + i * CHUNK
    pltpu.sync_copy(srci.at[pl.ds(base, CHUNK)], vsrc)
    pltpu.sync_copy(dsti.at[pl.ds(base, CHUNK)], vdst2)
    pltpu.async_copy(h.at[vsrc], vrows, sem).wait()
    pltpu.sync_copy(vrows, acc.at[vdst2], add=True)
    return carry

  lax.fori_loop(0, n_chunks, ebody, 0)
  plsc.subcore_barrier()

  def obody(dst_hbm, j, carry):
    r = s * SLAB + j * CHUNK
    pltpu.sync_copy(acc.at[pl.ds(r, CHUNK)], vrows)
    pltpu.sync_copy(vrows, dst_hbm.at[c, pl.ds(r, CHUNK)])
    return carry

  lax.fori_loop(0, SLAB // CHUNK, functools.partial(obody, out), 0)

  if with_cnt:
    plsc.subcore_barrier()
    pltpu.sync_copy(zrows, vrows)
    lax.fori_loop(0, SLAB // CHUNK, zbody, 0)
    plsc.subcore_barrier()

    def cbody(i, carry):
      base = t * ept + i * CHUNK
      pltpu.sync_copy(dsti.at[pl.ds(base, CHUNK)], vdst2)
      pltpu.sync_copy(vone, acc.at[vdst2], add=True)
      return carry

    lax.fori_loop(0, n_chunks, cbody, 0)
    plsc.subcore_barrier()
    lax.fori_loop(0, SLAB // CHUNK, functools.partial(obody, cntout), 0)


@functools.lru_cache(maxsize=None)
def _build_agg(epad, with_cnt):
  ept = epad // NW
  n_chunks = ept // CHUNK
  out_type = [jax.ShapeDtypeStruct((NC, NPAD, D), f32)]
  scratch = [
      pltpu.VMEM((CHUNK,), jnp.int32),
      pltpu.VMEM((CHUNK,), jnp.int32),
      pltpu.VMEM((CHUNK, D), f32),
  ]
  if with_cnt:
    out_type.append(jax.ShapeDtypeStruct((NC, NPAD, D), f32))
    scratch.append(pltpu.VMEM((CHUNK, D), f32))
  scratch.append(pltpu.VMEM_SHARED((NPAD, D), f32))
  scratch.append(pltpu.SemaphoreType.DMA)
  mesh = plsc.VectorSubcoreMesh(core_axis_name="c", subcore_axis_name="s")
  return pl.kernel(
      functools.partial(_sc_body, with_cnt, n_chunks, ept),
      out_type=out_type,
      mesh=mesh,
      scratch_types=scratch,
  )


def _tc_body(relu, p_ref, cnt_ref, h_ref, w_ref, b_ref, o_ref):
  s = p_ref[0] + p_ref[1] + h_ref[...]
  mean = s / cnt_ref[...]
  y = lax.dot_general(mean, w_ref[...], (((1,), (1,)), ((), ())),
                      preferred_element_type=f32) + b_ref[...]
  if relu:
    y = jnp.maximum(y, 0.0)
  o_ref[...] = y


BR = 2000


@functools.lru_cache(maxsize=None)
def _build_layer(relu):
  return pl.pallas_call(
      functools.partial(_tc_body, relu),
      grid=(N // BR,),
      in_specs=[
          pl.BlockSpec((NC, BR, D), lambda g: (0, g, 0)),
          pl.BlockSpec((BR, 1), lambda g: (g, 0)),
          pl.BlockSpec((BR, D), lambda g: (g, 0)),
          pl.BlockSpec((D, D), lambda g: (0, 0)),
          pl.BlockSpec((1, D), lambda g: (0, 0)),
      ],
      out_specs=pl.BlockSpec((BR, D), lambda g: (g, 0)),
      out_shape=jax.ShapeDtypeStruct((N, D), f32),
  )


def kernel(x, edge_index, W1, b1, W2, b2):
  e = edge_index.shape[1]
  epad = -(-e // (NW * CHUNK)) * (NW * CHUNK)
  src = jnp.pad(edge_index[0].astype(jnp.int32), (0, epad - e))
  dst = jnp.pad(edge_index[1].astype(jnp.int32), (0, epad - e))
  dst = jnp.where(src == dst, TRASH, dst)
  zrows = jnp.zeros((CHUNK, D), f32)
  orows = jnp.ones((CHUNK, D), f32)

  agg1, cntg = _build_agg(epad, True)(x, src, dst, zrows, orows)
  cnt = (cntg[0, :N, 0] + cntg[1, :N, 0] + 1.0).reshape(N, 1)
  h1 = _build_layer(True)(agg1, cnt, x, W1, b1.reshape(1, D))
  (agg2,) = _build_agg(epad, False)(h1, src, dst, zrows)
  return _build_layer(False)(agg2, cnt, h1, W2, b2.reshape(1, D))
